# baseline (speedup 1.0000x reference)
import functools

import jax
import jax.numpy as jnp
from jax import lax
from jax.experimental import pallas as pl
from jax.experimental.pallas import tpu as pltpu
from jax.experimental.pallas import tpu_sc as plsc

L = 16
NC = 2
NS = 16
NW = NC * NS

_GDN = lax.GatherDimensionNumbers(
    offset_dims=(), collapsed_slice_dims=(0,), start_index_map=(0,))


def _gather16(v, idx):
    return lax.gather(v, idx[:, None], dimension_numbers=_GDN,
                      slice_sizes=(1,),
                      mode=lax.GatherScatterMode.PROMISE_IN_BOUNDS)


@functools.lru_cache(maxsize=None)
def _build(npos, block_p, unroll):
    assert npos % NW == 0
    pos_w = npos // NW
    assert pos_w % block_p == 0
    nblk = pos_w // block_p
    steps = block_p // L

    mesh = plsc.VectorSubcoreMesh(
        core_axis_name="c", subcore_axis_name="s",
        num_cores=NC, num_subcores=NS)

    @functools.partial(
        pl.kernel,
        out_type=jax.ShapeDtypeStruct((NW * 2 * L,), jnp.float32),
        mesh=mesh,
        scratch_types=[
            pltpu.VMEM((block_p,), jnp.int32),
            pltpu.VMEM((block_p,), jnp.int32),
            pltpu.VMEM((2 * block_p,), jnp.float32),
            pltpu.VMEM((2 * block_p,), jnp.float32),
            pltpu.VMEM((2 * block_p,), jnp.float32),
            pltpu.VMEM((2 * block_p,), jnp.float32),
            pltpu.VMEM((2 * L,), jnp.float32),
            pltpu.SemaphoreType.DMA,
            pltpu.SemaphoreType.DMA,
        ],
    )
    def sc_kernel(regr_hbm, gt_hbm, mask_hbm, out_hbm,
                  mask_v0, mask_v1, regr_v0, regr_v1, gt_v0, gt_v1,
                  out_v, sem0, sem1):
        wid = lax.axis_index("s") * NC + lax.axis_index("c")
        pbase = wid * pos_w
        sems = (sem0, sem1)
        mask_b = (mask_v0, mask_v1)
        regr_b = (regr_v0, regr_v1)
        gt_b = (gt_v0, gt_v1)

        def start_block(blk, buf):
            p0 = pbase + blk * block_p
            cs = (
                pltpu.make_async_copy(
                    mask_hbm.at[pl.ds(p0, block_p)], mask_b[buf], sems[buf]),
                pltpu.make_async_copy(
                    regr_hbm.at[pl.ds(2 * p0, 2 * block_p)], regr_b[buf],
                    sems[buf]),
                pltpu.make_async_copy(
                    gt_hbm.at[pl.ds(2 * p0, 2 * block_p)], gt_b[buf],
                    sems[buf]),
            )
            for c in cs:
                c.start()
            return cs

        idx_lo = lax.shift_right_logical(lax.iota(jnp.int32, L), 1)
        idx_hi = idx_lo + (L // 2)

        nacc = jnp.zeros((L,), jnp.float32)
        cacc = jnp.zeros((L,), jnp.float32)

        inflight = {0: start_block(0, 0)}
        for blk in range(nblk):
            buf = blk & 1
            if blk + 1 < nblk:
                inflight[blk + 1] = start_block(blk + 1, (blk + 1) & 1)
            for c in inflight.pop(blk):
                c.wait()
            mv = mask_b[buf]
            rv = regr_b[buf]
            gv = gt_b[buf]

            @plsc.parallel_loop(0, steps, unroll=unroll, carry=(nacc, cacc))
            def _body(s, carry):
                na, ca = carry
                m = mv[pl.ds(s * L, L)].astype(jnp.float32)
                ca = ca + m
                e0 = _gather16(m, idx_lo)
                e1 = _gather16(m, idx_hi)
                r0 = rv[pl.ds(s * (2 * L), L)]
                r1 = rv[pl.ds(s * (2 * L) + L, L)]
                g0 = gv[pl.ds(s * (2 * L), L)]
                g1 = gv[pl.ds(s * (2 * L) + L, L)]
                na = na + jnp.abs(r0 - g0) * e0
                na = na + jnp.abs(r1 - g1) * e1
                return na, ca

            nacc, cacc = _body

        out_v[pl.ds(0, L)] = nacc
        out_v[pl.ds(L, L)] = cacc
        pltpu.sync_copy(out_v, out_hbm.at[pl.ds(wid * 2 * L, 2 * L)])

    return sc_kernel


def kernel(regr, gt_regr, mask):
    b, s, c = regr.shape
    npos = b * s
    regr_f = regr.reshape(npos * c)
    gt_f = gt_regr.reshape(npos * c)
    mask_f = mask.reshape(npos)
    parts = _build(npos, 8000, 4)(regr_f, gt_f, mask_f).reshape(NW, 2 * L)
    nsum = jnp.sum(parts[:, :L])
    csum = jnp.sum(parts[:, L:])
    return nsum / (csum * 2.0 + 0.0001)


# speedup vs baseline: 106.6215x; 106.6215x over previous
import functools

import jax
import jax.numpy as jnp
from jax import lax
from jax.experimental import pallas as pl
from jax.experimental.pallas import tpu as pltpu
from jax.experimental.pallas import tpu_sc as plsc

L = 16
NC = 2
NS = 16
NW = NC * NS


@functools.lru_cache(maxsize=None)
def _build(n_s, n_b, block_s, unroll):
    assert n_b % L == 0
    kb = n_b // L
    assert n_s % NW == 0
    s_w = n_s // NW
    assert s_w % block_s == 0
    nblk = s_w // block_s
    mlen = block_s * n_b
    rlen = 2 * mlen

    mesh = plsc.VectorSubcoreMesh(
        core_axis_name="c", subcore_axis_name="s",
        num_cores=NC, num_subcores=NS)

    @functools.partial(
        pl.kernel,
        out_type=jax.ShapeDtypeStruct((NW * 2 * L,), jnp.float32),
        mesh=mesh,
        scratch_types=[
            pltpu.VMEM((mlen,), jnp.int32),
            pltpu.VMEM((mlen,), jnp.int32),
            pltpu.VMEM((rlen,), jnp.float32),
            pltpu.VMEM((rlen,), jnp.float32),
            pltpu.VMEM((rlen,), jnp.float32),
            pltpu.VMEM((rlen,), jnp.float32),
            pltpu.VMEM((2 * L,), jnp.float32),
            pltpu.SemaphoreType.DMA,
            pltpu.SemaphoreType.DMA,
        ],
    )
    def sc_kernel(regr_hbm, gt_hbm, mask_hbm, out_hbm,
                  mask_v0, mask_v1, regr_v0, regr_v1, gt_v0, gt_v1,
                  out_v, sem0, sem1):
        wid = lax.axis_index("s") * NC + lax.axis_index("c")
        sbase = wid * s_w
        sems = (sem0, sem1)
        mask_b = (mask_v0, mask_v1)
        regr_b = (regr_v0, regr_v1)
        gt_b = (gt_v0, gt_v1)

        def start_block(blk, buf):
            s0 = sbase + blk * block_s
            cs = (
                pltpu.make_async_copy(
                    mask_hbm.at[pl.ds(s0 * n_b, mlen)], mask_b[buf],
                    sems[buf]),
                pltpu.make_async_copy(
                    regr_hbm.at[pl.ds(s0 * 2 * n_b, rlen)], regr_b[buf],
                    sems[buf]),
                pltpu.make_async_copy(
                    gt_hbm.at[pl.ds(s0 * 2 * n_b, rlen)], gt_b[buf],
                    sems[buf]),
            )
            for c in cs:
                c.start()
            return cs

        nacc = jnp.zeros((L,), jnp.float32)
        cacc = jnp.zeros((L,), jnp.float32)

        inflight = {0: start_block(0, 0)}
        for blk in range(nblk):
            buf = blk & 1
            if blk + 1 < nblk:
                inflight[blk + 1] = start_block(blk + 1, (blk + 1) & 1)
            for c in inflight.pop(blk):
                c.wait()
            mv = mask_b[buf]
            rv = regr_b[buf]
            gv = gt_b[buf]

            @plsc.parallel_loop(0, block_s, unroll=unroll,
                                carry=(nacc, cacc))
            def _body(s, carry):
                na, ca = carry
                mo = s * n_b
                ro = s * 2 * n_b
                for k in range(kb):
                    m = mv[pl.ds(mo + k * L, L)].astype(jnp.float32)
                    ca = ca + m
                    r0 = rv[pl.ds(ro + k * L, L)]
                    g0 = gv[pl.ds(ro + k * L, L)]
                    r1 = rv[pl.ds(ro + n_b + k * L, L)]
                    g1 = gv[pl.ds(ro + n_b + k * L, L)]
                    na = na + jnp.abs(r0 - g0) * m
                    na = na + jnp.abs(r1 - g1) * m
                return na, ca

            nacc, cacc = _body

        out_v[pl.ds(0, L)] = nacc
        out_v[pl.ds(L, L)] = cacc
        pltpu.sync_copy(out_v, out_hbm.at[pl.ds(wid * 2 * L, 2 * L)])

    return sc_kernel


def kernel(regr, gt_regr, mask):
    b, n_s, c = regr.shape
    regr_f = jnp.transpose(regr, (1, 2, 0)).reshape(-1)
    gt_f = jnp.transpose(gt_regr, (1, 2, 0)).reshape(-1)
    mask_f = jnp.transpose(mask, (1, 0)).reshape(-1)
    parts = _build(n_s, b, 25, 1)(regr_f, gt_f, mask_f).reshape(NW, 2 * L)
    nsum = jnp.sum(parts[:, :L])
    csum = jnp.sum(parts[:, L:])
    return nsum / (csum * 2.0 + 0.0001)


# speedup vs baseline: 124.7064x; 1.1696x over previous
import functools

import jax
import jax.numpy as jnp
from jax import lax
from jax.experimental import pallas as pl
from jax.experimental.pallas import tpu as pltpu
from jax.experimental.pallas import tpu_sc as plsc

L = 16
NC = 2
NS = 16
NW = NC * NS


@functools.lru_cache(maxsize=None)
def _build(n_s, n_b, block_s, unroll):
    assert n_b % L == 0
    kb = n_b // L
    assert n_s % NW == 0
    s_w = n_s // NW
    assert s_w % block_s == 0
    nblk = s_w // block_s
    mlen = block_s * n_b
    rlen = 2 * mlen

    nbuf = 3

    mesh = plsc.VectorSubcoreMesh(
        core_axis_name="c", subcore_axis_name="s",
        num_cores=NC, num_subcores=NS)

    @functools.partial(
        pl.kernel,
        out_type=jax.ShapeDtypeStruct((NW * 2 * L,), jnp.float32),
        mesh=mesh,
        scratch_types=(
            [pltpu.VMEM((mlen,), jnp.int32) for _ in range(nbuf)]
            + [pltpu.VMEM((rlen,), jnp.float32) for _ in range(2 * nbuf)]
            + [pltpu.VMEM((2 * L,), jnp.float32)]
            + [pltpu.SemaphoreType.DMA for _ in range(nbuf)]
        ),
    )
    def sc_kernel(regr_hbm, gt_hbm, mask_hbm, out_hbm, *scratch):
        mask_b = scratch[:nbuf]
        regr_b = scratch[nbuf:2 * nbuf]
        gt_b = scratch[2 * nbuf:3 * nbuf]
        out_v = scratch[3 * nbuf]
        sems = scratch[3 * nbuf + 1:]
        wid = lax.axis_index("s") * NC + lax.axis_index("c")
        sbase = wid * s_w

        def start_block(blk, buf):
            s0 = sbase + blk * block_s
            cs = (
                pltpu.make_async_copy(
                    mask_hbm.at[pl.ds(s0 * n_b, mlen)], mask_b[buf],
                    sems[buf]),
                pltpu.make_async_copy(
                    regr_hbm.at[pl.ds(s0 * 2 * n_b, rlen)], regr_b[buf],
                    sems[buf]),
                pltpu.make_async_copy(
                    gt_hbm.at[pl.ds(s0 * 2 * n_b, rlen)], gt_b[buf],
                    sems[buf]),
            )
            for c in cs:
                c.start()
            return cs

        zeros = jnp.zeros((L,), jnp.float32)
        naccs = (zeros,) * 4
        caccs = (zeros,) * 2

        inflight = {}
        for b in range(min(nbuf, nblk)):
            inflight[b] = start_block(b, b)
        for blk in range(nblk):
            buf = blk % nbuf
            for c in inflight.pop(blk):
                c.wait()
            mv = mask_b[buf]
            rv = regr_b[buf]
            gv = gt_b[buf]

            @plsc.parallel_loop(0, block_s, unroll=unroll,
                                carry=(naccs, caccs))
            def _body(s, carry):
                na, ca = carry
                na = list(na)
                ca = list(ca)
                mo = s * n_b
                ro = s * 2 * n_b
                for k in range(kb):
                    m = mv[pl.ds(mo + k * L, L)].astype(jnp.float32)
                    ca[k % 2] = ca[k % 2] + m
                    r0 = rv[pl.ds(ro + k * L, L)]
                    g0 = gv[pl.ds(ro + k * L, L)]
                    r1 = rv[pl.ds(ro + n_b + k * L, L)]
                    g1 = gv[pl.ds(ro + n_b + k * L, L)]
                    d = jnp.abs(r0 - g0) + jnp.abs(r1 - g1)
                    na[k % 4] = na[k % 4] + d * m
                return tuple(na), tuple(ca)

            naccs, caccs = _body
            blk_next = blk + nbuf
            if blk_next < nblk:
                inflight[blk_next] = start_block(blk_next, buf)

        out_v[pl.ds(0, L)] = (
            (naccs[0] + naccs[1]) + (naccs[2] + naccs[3]))
        out_v[pl.ds(L, L)] = caccs[0] + caccs[1]
        pltpu.sync_copy(out_v, out_hbm.at[pl.ds(wid * 2 * L, 2 * L)])

    return sc_kernel


def kernel(regr, gt_regr, mask):
    b, n_s, c = regr.shape
    regr_f = jnp.transpose(regr, (1, 2, 0)).reshape(-1)
    gt_f = jnp.transpose(gt_regr, (1, 2, 0)).reshape(-1)
    mask_f = jnp.transpose(mask, (1, 0)).reshape(-1)
    parts = _build(n_s, b, 25, 1)(regr_f, gt_f, mask_f).reshape(NW, 2 * L)
    nsum = jnp.sum(parts[:, :L])
    csum = jnp.sum(parts[:, L:])
    return nsum / (csum * 2.0 + 0.0001)
